# baseline (speedup 1.0000x reference)
import jax
import jax.numpy as jnp
from jax.experimental import pallas as pl
from jax.experimental.pallas import tpu as pltpu

B, L, D = 16, 4096, 256
H, O = 512, 128
CHUNK = 512
NC = L // CHUNK


def _num_chunks(length):
    return (length + CHUNK - 1) // CHUNK


def _fused_kernel(l1_ref, l2_ref,
                  x1_ref, x2_ref, len1f_ref, len2f_ref,
                  w1_ref, b1_ref, w2_ref, b2_ref,
                  out_ref, e1_ref, e2_ref):
    i = pl.program_id(0)
    j = pl.program_id(1)
    base = j * CHUNK
    row = jax.lax.broadcasted_iota(jnp.int32, (CHUNK, D), 0) + base

    def accum(len_ref, x_ref, e_ref):
        length = len_ref[i]

        @pl.when(base < length)
        def _():
            m = (row < length).astype(jnp.float32)
            part = jnp.sum(x_ref[0] * m, axis=0, keepdims=True)

            @pl.when(j == 0)
            def _():
                e_ref[pl.ds(i, 1), :] = part

            @pl.when(j > 0)
            def _():
                e_ref[pl.ds(i, 1), :] += part

    accum(l1_ref, x1_ref, e1_ref)
    accum(l2_ref, x2_ref, e2_ref)

    @pl.when((i == B - 1) & (j == NC - 1))
    def _():
        e1 = e1_ref[...] / len1f_ref[...]
        e2 = e2_ref[...] / len2f_ref[...]
        cat = jnp.concatenate([e1, e2, jnp.abs(e1 - e2), e1 * e2], axis=1)
        h = jnp.dot(cat, w1_ref[...], preferred_element_type=jnp.float32)
        h = jnp.maximum(h + b1_ref[...], 0.0)
        out_ref[...] = (
            jnp.dot(h, w2_ref[...], preferred_element_type=jnp.float32)
            + b2_ref[...]
        )


def kernel(X1, x1_lengths, X2, x2_lengths, W1, b1, W2, b2):
    len1f = x1_lengths.astype(jnp.float32).reshape(B, 1)
    len2f = x2_lengths.astype(jnp.float32).reshape(B, 1)

    def x_spec(which):
        def index_map(i, j, l1, l2):
            lens = l1 if which == 0 else l2
            return (i, jnp.minimum(j, _num_chunks(lens[i]) - 1), 0)
        return pl.BlockSpec((1, CHUNK, D), index_map)

    const = lambda shape: pl.BlockSpec(shape, lambda i, j, l1, l2: (0,) * len(shape))

    grid_spec = pltpu.PrefetchScalarGridSpec(
        num_scalar_prefetch=2,
        grid=(B, NC),
        in_specs=[
            x_spec(0),
            x_spec(1),
            const((B, 1)),
            const((B, 1)),
            const((4 * D, H)),
            const((1, H)),
            const((H, O)),
            const((1, O)),
        ],
        out_specs=const((B, O)),
        scratch_shapes=[
            pltpu.VMEM((B, D), jnp.float32),
            pltpu.VMEM((B, D), jnp.float32),
        ],
    )

    return pl.pallas_call(
        _fused_kernel,
        grid_spec=grid_spec,
        out_shape=jax.ShapeDtypeStruct((B, O), jnp.float32),
        compiler_params=pltpu.CompilerParams(
            dimension_semantics=("arbitrary", "arbitrary"),
        ),
    )(x1_lengths, x2_lengths, X1, X2, len1f, len2f,
      W1, b1.reshape(1, H), W2, b2.reshape(1, O))


# speedup vs baseline: 1.0077x; 1.0077x over previous
import jax
import jax.numpy as jnp
from jax.experimental import pallas as pl
from jax.experimental.pallas import tpu as pltpu

B, L, D = 16, 4096, 256
H, O = 512, 128
CHUNK = 512
NC = L // CHUNK


def _num_chunks(length):
    return (length + CHUNK - 1) // CHUNK


def _fused_kernel(l1_ref, l2_ref,
                  x1_ref, x2_ref, len1f_ref, len2f_ref,
                  w1_ref, b1_ref, w2_ref, b2_ref,
                  out_ref, e1_ref, e2_ref):
    i = pl.program_id(0)
    j = pl.program_id(1)
    base = j * CHUNK
    row = jax.lax.broadcasted_iota(jnp.int32, (1, CHUNK), 1) + base

    def accum(len_ref, x_ref, e_ref):
        length = len_ref[i]

        @pl.when(base < length)
        def _():
            m = (row < length).astype(jnp.float32)
            part = jnp.dot(m, x_ref[0], preferred_element_type=jnp.float32)

            @pl.when(j == 0)
            def _():
                e_ref[pl.ds(i, 1), :] = part

            @pl.when(j > 0)
            def _():
                e_ref[pl.ds(i, 1), :] += part

    accum(l1_ref, x1_ref, e1_ref)
    accum(l2_ref, x2_ref, e2_ref)

    @pl.when((i == B - 1) & (j == NC - 1))
    def _():
        e1 = e1_ref[...] / len1f_ref[...]
        e2 = e2_ref[...] / len2f_ref[...]
        cat = jnp.concatenate([e1, e2, jnp.abs(e1 - e2), e1 * e2], axis=1)
        h = jnp.dot(cat, w1_ref[...], preferred_element_type=jnp.float32)
        h = jnp.maximum(h + b1_ref[...], 0.0)
        out_ref[...] = (
            jnp.dot(h, w2_ref[...], preferred_element_type=jnp.float32)
            + b2_ref[...]
        )


def kernel(X1, x1_lengths, X2, x2_lengths, W1, b1, W2, b2):
    len1f = x1_lengths.astype(jnp.float32).reshape(B, 1)
    len2f = x2_lengths.astype(jnp.float32).reshape(B, 1)

    def x_spec(which):
        def index_map(i, j, l1, l2):
            lens = l1 if which == 0 else l2
            return (i, jnp.minimum(j, _num_chunks(lens[i]) - 1), 0)
        return pl.BlockSpec((1, CHUNK, D), index_map)

    const = lambda shape: pl.BlockSpec(shape, lambda i, j, l1, l2: (0,) * len(shape))

    grid_spec = pltpu.PrefetchScalarGridSpec(
        num_scalar_prefetch=2,
        grid=(B, NC),
        in_specs=[
            x_spec(0),
            x_spec(1),
            const((B, 1)),
            const((B, 1)),
            const((4 * D, H)),
            const((1, H)),
            const((H, O)),
            const((1, O)),
        ],
        out_specs=const((B, O)),
        scratch_shapes=[
            pltpu.VMEM((B, D), jnp.float32),
            pltpu.VMEM((B, D), jnp.float32),
        ],
    )

    return pl.pallas_call(
        _fused_kernel,
        grid_spec=grid_spec,
        out_shape=jax.ShapeDtypeStruct((B, O), jnp.float32),
        compiler_params=pltpu.CompilerParams(
            dimension_semantics=("arbitrary", "arbitrary"),
        ),
    )(x1_lengths, x2_lengths, X1, X2, len1f, len2f,
      W1, b1.reshape(1, H), W2, b2.reshape(1, O))


# speedup vs baseline: 1.0439x; 1.0359x over previous
import jax
import jax.numpy as jnp
from jax.experimental import pallas as pl
from jax.experimental.pallas import tpu as pltpu

B, L, D = 16, 4096, 256
H, O = 512, 128
CHUNK = 512
NC = L // CHUNK


def _num_chunks(length):
    return (length + CHUNK - 1) // CHUNK


def _rowsum8(x):
    n = x.shape[0]
    while n > 8:
        n //= 2
        x = x[:n] + x[n:]
    return x


def _fused_kernel(l1_ref, l2_ref,
                  x1_ref, x2_ref, len1f_ref, len2f_ref,
                  w1_ref, b1_ref, w2_ref, b2_ref,
                  out_ref, acc1_ref, acc2_ref, e1_ref, e2_ref):
    i = pl.program_id(0)
    j = pl.program_id(1)
    base = j * CHUNK

    def accum(len_ref, x_ref, acc_ref, e_ref):
        length = len_ref[i]

        @pl.when(j == 0)
        def _():
            acc_ref[...] = jnp.zeros_like(acc_ref)

        @pl.when(base + CHUNK <= length)
        def _():
            acc_ref[...] += _rowsum8(x_ref[0])

        @pl.when((base < length) & (length < base + CHUNK))
        def _():
            row = jax.lax.broadcasted_iota(jnp.int32, (CHUNK, D), 0) + base
            xm = jnp.where(row < length, x_ref[0], 0.0)
            acc_ref[...] += _rowsum8(xm)

        @pl.when(j == NC - 1)
        def _():
            e_ref[pl.ds(i, 1), :] = jnp.sum(acc_ref[...], axis=0,
                                            keepdims=True)

    accum(l1_ref, x1_ref, acc1_ref, e1_ref)
    accum(l2_ref, x2_ref, acc2_ref, e2_ref)

    @pl.when((i == B - 1) & (j == NC - 1))
    def _():
        e1 = e1_ref[...] / len1f_ref[...]
        e2 = e2_ref[...] / len2f_ref[...]
        cat = jnp.concatenate([e1, e2, jnp.abs(e1 - e2), e1 * e2], axis=1)
        h = jnp.dot(cat, w1_ref[...], preferred_element_type=jnp.float32)
        h = jnp.maximum(h + b1_ref[...], 0.0)
        out_ref[...] = (
            jnp.dot(h, w2_ref[...], preferred_element_type=jnp.float32)
            + b2_ref[...]
        )


def kernel(X1, x1_lengths, X2, x2_lengths, W1, b1, W2, b2):
    len1f = x1_lengths.astype(jnp.float32).reshape(B, 1)
    len2f = x2_lengths.astype(jnp.float32).reshape(B, 1)

    def x_spec(which):
        def index_map(i, j, l1, l2):
            lens = l1 if which == 0 else l2
            return (i, jnp.minimum(j, _num_chunks(lens[i]) - 1), 0)
        return pl.BlockSpec((1, CHUNK, D), index_map)

    const = lambda shape: pl.BlockSpec(shape, lambda i, j, l1, l2: (0,) * len(shape))

    grid_spec = pltpu.PrefetchScalarGridSpec(
        num_scalar_prefetch=2,
        grid=(B, NC),
        in_specs=[
            x_spec(0),
            x_spec(1),
            const((B, 1)),
            const((B, 1)),
            const((4 * D, H)),
            const((1, H)),
            const((H, O)),
            const((1, O)),
        ],
        out_specs=const((B, O)),
        scratch_shapes=[
            pltpu.VMEM((8, D), jnp.float32),
            pltpu.VMEM((8, D), jnp.float32),
            pltpu.VMEM((B, D), jnp.float32),
            pltpu.VMEM((B, D), jnp.float32),
        ],
    )

    return pl.pallas_call(
        _fused_kernel,
        grid_spec=grid_spec,
        out_shape=jax.ShapeDtypeStruct((B, O), jnp.float32),
        compiler_params=pltpu.CompilerParams(
            dimension_semantics=("arbitrary", "arbitrary"),
        ),
    )(x1_lengths, x2_lengths, X1, X2, len1f, len2f,
      W1, b1.reshape(1, H), W2, b2.reshape(1, O))


# speedup vs baseline: 1.3830x; 1.3249x over previous
import jax
import jax.numpy as jnp
from jax.experimental import pallas as pl
from jax.experimental.pallas import tpu as pltpu

B, L, D = 16, 4096, 256
H, O = 512, 128
CHUNK = 1024
NC = L // CHUNK


def _num_chunks(length):
    return (length + CHUNK - 1) // CHUNK


def _rowsum8(x):
    n = x.shape[0]
    while n > 8:
        n //= 2
        x = x[:n] + x[n:]
    return x


def _fused_kernel(l1_ref, l2_ref,
                  x1_ref, x2_ref, len1f_ref, len2f_ref,
                  w1_ref, b1_ref, w2_ref, b2_ref,
                  out_ref, acc1_ref, acc2_ref, e1_ref, e2_ref):
    i = pl.program_id(0)
    j = pl.program_id(1)
    base = j * CHUNK

    def accum(len_ref, x_ref, acc_ref, e_ref):
        length = len_ref[i]

        @pl.when(j == 0)
        def _():
            acc_ref[...] = jnp.zeros_like(acc_ref)

        @pl.when(base + CHUNK <= length)
        def _():
            acc_ref[...] += _rowsum8(x_ref[0])

        @pl.when((base < length) & (length < base + CHUNK))
        def _():
            row = jax.lax.broadcasted_iota(jnp.int32, (CHUNK, D), 0) + base
            xm = jnp.where(row < length, x_ref[0], 0.0)
            acc_ref[...] += _rowsum8(xm)

        @pl.when(j == NC - 1)
        def _():
            e_ref[pl.ds(i, 1), :] = jnp.sum(acc_ref[...], axis=0,
                                            keepdims=True)

    accum(l1_ref, x1_ref, acc1_ref, e1_ref)
    accum(l2_ref, x2_ref, acc2_ref, e2_ref)

    @pl.when((i == B - 1) & (j == NC - 1))
    def _():
        e1 = e1_ref[...] / len1f_ref[...]
        e2 = e2_ref[...] / len2f_ref[...]
        cat = jnp.concatenate([e1, e2, jnp.abs(e1 - e2), e1 * e2], axis=1)
        h = jnp.dot(cat, w1_ref[...], preferred_element_type=jnp.float32)
        h = jnp.maximum(h + b1_ref[...], 0.0)
        out_ref[...] = (
            jnp.dot(h, w2_ref[...], preferred_element_type=jnp.float32)
            + b2_ref[...]
        )


def kernel(X1, x1_lengths, X2, x2_lengths, W1, b1, W2, b2):
    len1f = x1_lengths.astype(jnp.float32).reshape(B, 1)
    len2f = x2_lengths.astype(jnp.float32).reshape(B, 1)

    def x_spec(which):
        def index_map(i, j, l1, l2):
            lens = l1 if which == 0 else l2
            return (i, jnp.minimum(j, _num_chunks(lens[i]) - 1), 0)
        return pl.BlockSpec((1, CHUNK, D), index_map)

    const = lambda shape: pl.BlockSpec(shape, lambda i, j, l1, l2: (0,) * len(shape))

    grid_spec = pltpu.PrefetchScalarGridSpec(
        num_scalar_prefetch=2,
        grid=(B, NC),
        in_specs=[
            x_spec(0),
            x_spec(1),
            const((B, 1)),
            const((B, 1)),
            const((4 * D, H)),
            const((1, H)),
            const((H, O)),
            const((1, O)),
        ],
        out_specs=const((B, O)),
        scratch_shapes=[
            pltpu.VMEM((8, D), jnp.float32),
            pltpu.VMEM((8, D), jnp.float32),
            pltpu.VMEM((B, D), jnp.float32),
            pltpu.VMEM((B, D), jnp.float32),
        ],
    )

    return pl.pallas_call(
        _fused_kernel,
        grid_spec=grid_spec,
        out_shape=jax.ShapeDtypeStruct((B, O), jnp.float32),
        compiler_params=pltpu.CompilerParams(
            dimension_semantics=("arbitrary", "arbitrary"),
        ),
    )(x1_lengths, x2_lengths, X1, X2, len1f, len2f,
      W1, b1.reshape(1, H), W2, b2.reshape(1, O))
